# baseline (speedup 1.0000x reference)
import jax
import jax.numpy as jnp
from jax import lax
from jax.experimental import pallas as pl
from jax.experimental.pallas import tpu as pltpu
from jax.experimental.pallas import tpu_sc as plsc

_NF = 26
_FD = 16
_NPAIR = (_NF * (_NF - 1)) // 2
_ROW_IN = _NF * _FD
_ROW_OUT = _NPAIR * 3 * _FD
_BATCH = 4096
_NW = 32
_LANES = _BATCH // _NW
_NSUB = _LANES // 16
_NBUF = 4


def _pin_body(xt_hbm, out_hbm, xblk, obuf0, obuf1, obuf2, obuf3,
              sem0, sem1, sem2, sem3):
    wid = lax.axis_index("s") * 2 + lax.axis_index("c")
    lane0 = wid * _LANES

    pltpu.sync_copy(xt_hbm.at[:, pl.ds(lane0, _LANES)], xblk)

    obufs = (obuf0, obuf1, obuf2, obuf3)
    sems = (sem0, sem1, sem2, sem3)

    def compute_pair(obuf, ir, jr):
        for c in range(_FD):
            for u in range(_NSUB):
                sl = pl.ds(16 * u, 16)
                av = xblk[ir + c, sl]
                bv = xblk[jr + c, sl]
                obuf[c, sl] = av
                obuf[_FD + c, sl] = bv
                obuf[2 * _FD + c, sl] = av * bv

    def seg(i, carry):
        seg_base = (i * (2 * _NF - 1 - i)) // 2

        def pairj(j, carry2):
            p = seg_base + (j - i - 1)
            slot = lax.rem(p, _NBUF)
            ir = _FD * i
            jr = _FD * j
            for k in range(_NBUF):
                @pl.when(slot == k)
                def _run(k=k):
                    @pl.when(p >= _NBUF)
                    def _drain():
                        pltpu.make_async_copy(
                            obufs[k],
                            out_hbm.at[pl.ds(0, 3 * _FD), pl.ds(lane0, _LANES)],
                            sems[k]).wait()

                    compute_pair(obufs[k], ir, jr)
                    pltpu.async_copy(
                        obufs[k],
                        out_hbm.at[pl.ds(3 * _FD * p, 3 * _FD),
                                   pl.ds(lane0, _LANES)],
                        sems[k])
            return carry2

        return lax.fori_loop(i + 1, _NF, pairj, carry)

    lax.fori_loop(0, _NF - 1, seg, 0)

    for k in range(_NBUF):
        pltpu.make_async_copy(
            obufs[k],
            out_hbm.at[pl.ds(0, 3 * _FD), pl.ds(lane0, _LANES)],
            sems[k]).wait()


@jax.jit
def kernel(x):
    xt = x.transpose(1, 2, 0).reshape(_ROW_IN, _BATCH)
    run = pl.kernel(
        _pin_body,
        out_type=jax.ShapeDtypeStruct((_ROW_OUT, _BATCH), jnp.float32),
        scratch_types=[
            pltpu.VMEM((_ROW_IN, _LANES), jnp.float32),
            pltpu.VMEM((3 * _FD, _LANES), jnp.float32),
            pltpu.VMEM((3 * _FD, _LANES), jnp.float32),
            pltpu.VMEM((3 * _FD, _LANES), jnp.float32),
            pltpu.VMEM((3 * _FD, _LANES), jnp.float32),
            pltpu.SemaphoreType.DMA,
            pltpu.SemaphoreType.DMA,
            pltpu.SemaphoreType.DMA,
            pltpu.SemaphoreType.DMA,
        ],
        mesh=plsc.VectorSubcoreMesh(core_axis_name="c", subcore_axis_name="s"),
    )
    out_t = run(xt)
    return out_t.reshape(_NPAIR, 3 * _FD, _BATCH).transpose(2, 0, 1)


# speedup vs baseline: 1.6460x; 1.6460x over previous
import jax
import jax.numpy as jnp
from jax import lax
from jax.experimental import pallas as pl
from jax.experimental.pallas import tpu as pltpu
from jax.experimental.pallas import tpu_sc as plsc

_NF = 26
_FD = 16
_NPAIR = (_NF * (_NF - 1)) // 2
_ROW_IN = _NF * _FD
_ROW_OUT = _NPAIR * 3 * _FD
_BATCH = 4096
_NW = 32
_LANES = _BATCH // _NW
_NSUB = _LANES // 16


def _pin_body(xt_hbm, out_hbm, xblk, obuf, sem0, sem1):
    wid = lax.axis_index("s") * 2 + lax.axis_index("c")
    lane0 = wid * _LANES

    pltpu.sync_copy(xt_hbm.at[:, pl.ds(lane0, _LANES)], xblk)

    sems = (sem0, sem1)

    def seg(i, carry):
        seg_base = (i * (2 * _NF - 1 - i)) // 2

        def pairj(j, carry2):
            p = seg_base + (j - i - 1)
            slot = lax.rem(p, 2)
            ir = _FD * i
            jr = _FD * j

            for k in range(2):
                @pl.when((slot == k) & (p >= 2))
                def _drain(k=k):
                    pltpu.make_async_copy(
                        obuf.at[k],
                        out_hbm.at[pl.ds(0, 3 * _FD), pl.ds(lane0, _LANES)],
                        sems[k]).wait()

            for c in range(_FD):
                for u in range(_NSUB):
                    sl = pl.ds(16 * u, 16)
                    av = xblk[ir + c, sl]
                    bv = xblk[jr + c, sl]
                    obuf[slot, c, sl] = av
                    obuf[slot, _FD + c, sl] = bv
                    obuf[slot, 2 * _FD + c, sl] = av * bv

            for k in range(2):
                @pl.when(slot == k)
                def _issue(k=k):
                    pltpu.async_copy(
                        obuf.at[k],
                        out_hbm.at[pl.ds(3 * _FD * p, 3 * _FD),
                                   pl.ds(lane0, _LANES)],
                        sems[k])
            return carry2

        return lax.fori_loop(i + 1, _NF, pairj, carry)

    lax.fori_loop(0, _NF - 1, seg, 0)

    for k in range(2):
        pltpu.make_async_copy(
            obuf.at[k],
            out_hbm.at[pl.ds(0, 3 * _FD), pl.ds(lane0, _LANES)],
            sems[k]).wait()


@jax.jit
def kernel(x):
    xt = x.transpose(1, 2, 0).reshape(_ROW_IN, _BATCH)
    run = pl.kernel(
        _pin_body,
        out_type=jax.ShapeDtypeStruct((_ROW_OUT, _BATCH), jnp.float32),
        scratch_types=[
            pltpu.VMEM((_ROW_IN, _LANES), jnp.float32),
            pltpu.VMEM((2, 3 * _FD, _LANES), jnp.float32),
            pltpu.SemaphoreType.DMA,
            pltpu.SemaphoreType.DMA,
        ],
        mesh=plsc.VectorSubcoreMesh(core_axis_name="c", subcore_axis_name="s"),
    )
    out_t = run(xt)
    return out_t.reshape(_NPAIR, 3 * _FD, _BATCH).transpose(2, 0, 1)
